# baseline (speedup 1.0000x reference)
import functools

import jax
import jax.numpy as jnp
from jax import lax
from jax.experimental import pallas as pl
from jax.experimental.pallas import tpu as pltpu
from jax.experimental.pallas import tpu_sc as plsc

N, C = 512, 256
NUM_CORES = 2
NUM_SUBCORES = 16
NW = NUM_CORES * NUM_SUBCORES
RPW = N // NW
L = 16
NCHUNK = C // L

_LN2 = 0.6931471805599453
_LOG_COEF = (
    1.0,
    -0.500000536441803,
    0.3333345055580139,
    -0.24994175136089325,
    0.19982793927192688,
    -0.16819174587726593,
    0.14910875260829926,
    -0.11938634514808655,
    0.053567204624414444,
)


def _softlog(x):
    bits = plsc.bitcast(x, jnp.int32)
    eb = (bits + 0x00400000) >> 23
    scale = plsc.bitcast((254 - eb) << 23, jnp.float32)
    t = x * scale - 1.0
    p = jnp.full_like(t, _LOG_COEF[-1])
    for coef in _LOG_COEF[-2::-1]:
        p = p * t + coef
    return (eb - 127).astype(jnp.float32) * _LN2 + t * p


def _bfly(v, op):
    for k in (8, 4, 2, 1):
        idx = lax.iota(jnp.int32, L) ^ k
        v = op(v, jnp.take_along_axis(v, idx, axis=0))
    return v


def _sc_partials(student, teacher, target):
    mesh = plsc.VectorSubcoreMesh(core_axis_name="c", subcore_axis_name="s")

    @functools.partial(
        pl.kernel,
        out_type=jax.ShapeDtypeStruct((NW, L), jnp.float32),
        mesh=mesh,
        compiler_params=pltpu.CompilerParams(needs_layout_passes=False),
        scratch_types=[
            pltpu.VMEM((RPW, C), jnp.float32),
            pltpu.VMEM((RPW, C), jnp.float32),
            pltpu.VMEM((RPW, C), jnp.float32),
            pltpu.VMEM((C,), jnp.float32),
            pltpu.VMEM((C,), jnp.float32),
            pltpu.VMEM((C,), jnp.float32),
            pltpu.VMEM((L,), jnp.float32),
        ],
    )
    def body(s_hbm, t_hbm, g_hbm, out_hbm, s_v, t_v, g_v, es_v, et_v, d_v,
             acc_v):
        wid = lax.axis_index("s") * NUM_CORES + lax.axis_index("c")
        base = wid * RPW
        pltpu.sync_copy(s_hbm.at[pl.ds(base, RPW)], s_v)
        pltpu.sync_copy(t_hbm.at[pl.ds(base, RPW)], t_v)
        pltpu.sync_copy(g_hbm.at[pl.ds(base, RPW)], g_v)

        def row_body(i, acc):
            ms = [s_v[i, pl.ds(0, L)], s_v[i, pl.ds(L, L)]]
            mt = [t_v[i, pl.ds(0, L)], t_v[i, pl.ds(L, L)]]
            for j in range(2, NCHUNK):
                sl = pl.ds(j * L, L)
                k = j & 1
                ms[k] = jnp.maximum(ms[k], s_v[i, sl])
                mt[k] = jnp.maximum(mt[k], t_v[i, sl])
            M_s = _bfly(jnp.maximum(ms[0], ms[1]), jnp.maximum)
            M_t = _bfly(jnp.maximum(mt[0], mt[1]), jnp.maximum)

            zero = jnp.zeros((L,), jnp.float32)
            Esl = [zero, zero]
            Etl = [zero, zero]
            Avl = [zero, zero]
            for j in range(NCHUNK):
                sl = pl.ds(j * L, L)
                sv = s_v[i, sl]
                tv = t_v[i, sl]
                gm = 1.0 - g_v[i, sl]
                es = jnp.exp(sv - M_s)
                et = jnp.exp(tv - M_t)
                d = tv - sv
                es_v[sl] = es
                et_v[sl] = et
                d_v[sl] = d
                etg = et * gm
                k = j & 1
                Esl[k] = Esl[k] + es * gm
                Etl[k] = Etl[k] + etg
                Avl[k] = Avl[k] + etg * d
            Es = _bfly(Esl[0] + Esl[1], jnp.add)
            Et = _bfly(Etl[0] + Etl[1], jnp.add)
            A = _bfly(Avl[0] + Avl[1], jnp.add)
            Kv = M_s - M_t

            kls = [zero, zero]
            for j in range(NCHUNK):
                sl = pl.ds(j * L, L)
                es = es_v[sl]
                et = et_v[sl]
                d = d_v[sl]
                zs = Es + es
                zt = Et + et
                num = A + et * d
                rzt = 1.0 / zt
                kl = num * rzt + _softlog(zs * rzt) + Kv
                k = j & 1
                kls[k] = kls[k] + kl * g_v[i, sl]
            return acc + kls[0] + kls[1]

        acc = lax.fori_loop(0, RPW, row_body, jnp.zeros((L,), jnp.float32))
        acc_v[...] = acc * jnp.float32(1.0 / N)
        pltpu.sync_copy(acc_v, out_hbm.at[wid])

    return body(student, teacher, target)


def kernel(student, teacher, target):
    partials = _sc_partials(student, teacher, target)
    return jnp.sum(partials)


# speedup vs baseline: 1.1198x; 1.1198x over previous
import functools

import jax
import jax.numpy as jnp
from jax import lax
from jax.experimental import pallas as pl
from jax.experimental.pallas import tpu as pltpu
from jax.experimental.pallas import tpu_sc as plsc

N, C = 512, 256
NUM_CORES = 2
NUM_SUBCORES = 16
NW = NUM_CORES * NUM_SUBCORES
RPW = N // NW
L = 16
NCHUNK = C // L
CPI = 2

_LN2 = 0.6931471805599453
_LOG_COEF = (
    1.0000005960464478,
    -0.4999879002571106,
    0.33319392800331116,
    -0.25064709782600403,
    0.20591352880001068,
    -0.16623561084270477,
    0.08214391767978668,
)


def _softlog_biased(x):
    bits = plsc.bitcast(x, jnp.int32)
    eb = (bits + 0x00400000) >> 23
    scale = plsc.bitcast((254 - eb) << 23, jnp.float32)
    t = x * scale - 1.0
    p = jnp.full_like(t, _LOG_COEF[-1])
    for coef in _LOG_COEF[-2::-1]:
        p = p * t + coef
    return eb.astype(jnp.float32) * _LN2 + t * p


def _bfly_multi(vs, ops):
    for k in (8, 4, 2, 1):
        idx = lax.iota(jnp.int32, L) ^ k
        perms = [jnp.take_along_axis(v, idx, axis=0) for v in vs]
        vs = [op(v, pv) for v, pv, op in zip(vs, perms, ops)]
    return vs


def _sc_partials(student, teacher, target):
    mesh = plsc.VectorSubcoreMesh(core_axis_name="c", subcore_axis_name="s")

    @functools.partial(
        pl.kernel,
        out_type=jax.ShapeDtypeStruct((NW, L), jnp.float32),
        mesh=mesh,
        compiler_params=pltpu.CompilerParams(needs_layout_passes=False),
        scratch_types=[
            pltpu.VMEM((RPW, C), jnp.float32),
            pltpu.VMEM((RPW, C), jnp.float32),
            pltpu.VMEM((RPW, C), jnp.float32),
            pltpu.VMEM((C,), jnp.float32),
            pltpu.VMEM((C,), jnp.float32),
            pltpu.VMEM((C,), jnp.float32),
            pltpu.VMEM((L,), jnp.float32),
            pltpu.SemaphoreType.DMA,
        ],
    )
    def body(s_hbm, t_hbm, g_hbm, out_hbm, s_v, t_v, g_v, es_v, et_v, d_v,
             acc_v, sem):
        wid = lax.axis_index("s") * NUM_CORES + lax.axis_index("c")
        base = wid * RPW
        c1 = pltpu.async_copy(s_hbm.at[pl.ds(base, RPW)], s_v, sem)
        c2 = pltpu.async_copy(t_hbm.at[pl.ds(base, RPW)], t_v, sem)
        c3 = pltpu.async_copy(g_hbm.at[pl.ds(base, RPW)], g_v, sem)
        c1.wait()
        c2.wait()
        c3.wait()

        J = range(CPI)

        def row_body(i, acc):
            ms = [s_v[i, pl.ds(j * L, L)] for j in J]
            mt = [t_v[i, pl.ds(j * L, L)] for j in J]
            for jj in range(CPI, NCHUNK, CPI):
                svs = [s_v[i, pl.ds((jj + j) * L, L)] for j in J]
                tvs = [t_v[i, pl.ds((jj + j) * L, L)] for j in J]
                ms = [jnp.maximum(a, b) for a, b in zip(ms, svs)]
                mt = [jnp.maximum(a, b) for a, b in zip(mt, tvs)]
            M_s, M_t = _bfly_multi(
                [functools.reduce(jnp.maximum, ms),
                 functools.reduce(jnp.maximum, mt)],
                [jnp.maximum, jnp.maximum],
            )

            zero = jnp.zeros((L,), jnp.float32)
            Es, Et, Av = zero, zero, zero
            for jj in range(0, NCHUNK, CPI):
                sls = [pl.ds((jj + j) * L, L) for j in J]
                svs = [s_v[i, sl] for sl in sls]
                tvs = [t_v[i, sl] for sl in sls]
                gvs = [g_v[i, sl] for sl in sls]
                xs = [sv - M_s for sv in svs]
                xt = [tv - M_t for tv in tvs]
                ess = [jnp.exp(x) for x in xs]
                ets = [jnp.exp(x) for x in xt]
                ds = [tv - sv for sv, tv in zip(svs, tvs)]
                gms = [1.0 - gv for gv in gvs]
                for j in J:
                    es_v[sls[j]] = ess[j]
                    et_v[sls[j]] = ets[j]
                    d_v[sls[j]] = ds[j]
                etgs = [et * gm for et, gm in zip(ets, gms)]
                for j in J:
                    Es = Es + ess[j] * gms[j]
                    Et = Et + etgs[j]
                    Av = Av + etgs[j] * ds[j]
            Es, Et, A = _bfly_multi([Es, Et, Av], [jnp.add] * 3)
            Kv = (M_s - M_t) - jnp.float32(127.0 * _LN2)

            kacc = zero
            for jj in range(0, NCHUNK, CPI):
                sls = [pl.ds((jj + j) * L, L) for j in J]
                ess = [es_v[sl] for sl in sls]
                ets = [et_v[sl] for sl in sls]
                ds = [d_v[sl] for sl in sls]
                zss = [Es + es for es in ess]
                zts = [Et + et for et in ets]
                nums = [A + et * d for et, d in zip(ets, ds)]
                rzts = [1.0 / zt for zt in zts]
                largs = [zs * rzt for zs, rzt in zip(zss, rzts)]
                logs = [_softlog_biased(la) for la in largs]
                for j in J:
                    kl = nums[j] * rzts[j] + logs[j] + Kv
                    kacc = kacc + kl * g_v[i, sls[j]]
            return acc + kacc

        acc = lax.fori_loop(0, RPW, row_body, jnp.zeros((L,), jnp.float32))
        acc_v[...] = acc * jnp.float32(1.0 / N)
        pltpu.sync_copy(acc_v, out_hbm.at[wid])

    return body(student, teacher, target)


def kernel(student, teacher, target):
    partials = _sc_partials(student, teacher, target)
    return jnp.sum(partials)


# speedup vs baseline: 1.1765x; 1.0506x over previous
import functools

import jax
import jax.numpy as jnp
from jax import lax
from jax.experimental import pallas as pl
from jax.experimental.pallas import tpu as pltpu
from jax.experimental.pallas import tpu_sc as plsc

N, C = 512, 256
NUM_CORES = 2
NUM_SUBCORES = 16
NW = NUM_CORES * NUM_SUBCORES
RPW = N // NW
L = 16
NCHUNK = C // L
CPI = 16

_LN2 = 0.6931471805599453
_LOG_COEF = (
    1.0000005960464478,
    -0.4999879002571106,
    0.33319392800331116,
    -0.25064709782600403,
    0.20591352880001068,
    -0.16623561084270477,
    0.08214391767978668,
)


def _softlog_biased(x):
    bits = plsc.bitcast(x, jnp.int32)
    eb = (bits + 0x00400000) >> 23
    scale = plsc.bitcast((254 - eb) << 23, jnp.float32)
    t = x * scale - 1.0
    p = jnp.full_like(t, _LOG_COEF[-1])
    for coef in _LOG_COEF[-2::-1]:
        p = p * t + coef
    return eb.astype(jnp.float32) * _LN2 + t * p


def _bfly_multi(vs, ops):
    for k in (8, 4, 2, 1):
        idx = lax.iota(jnp.int32, L) ^ k
        perms = [jnp.take_along_axis(v, idx, axis=0) for v in vs]
        vs = [op(v, pv) for v, pv, op in zip(vs, perms, ops)]
    return vs


def _sc_partials(student, teacher, target):
    mesh = plsc.VectorSubcoreMesh(core_axis_name="c", subcore_axis_name="s")

    @functools.partial(
        pl.kernel,
        out_type=jax.ShapeDtypeStruct((NW, L), jnp.float32),
        mesh=mesh,
        compiler_params=pltpu.CompilerParams(needs_layout_passes=False),
        scratch_types=[
            pltpu.VMEM((RPW, C), jnp.float32),
            pltpu.VMEM((RPW, C), jnp.float32),
            pltpu.VMEM((RPW, C), jnp.float32),
            pltpu.VMEM((C,), jnp.float32),
            pltpu.VMEM((C,), jnp.float32),
            pltpu.VMEM((C,), jnp.float32),
            pltpu.VMEM((L,), jnp.float32),
            pltpu.SemaphoreType.DMA,
        ],
    )
    def body(s_hbm, t_hbm, g_hbm, out_hbm, s_v, t_v, g_v, es_v, et_v, d_v,
             acc_v, sem):
        wid = lax.axis_index("s") * NUM_CORES + lax.axis_index("c")
        base = wid * RPW
        c1 = pltpu.async_copy(s_hbm.at[pl.ds(base, RPW)], s_v, sem)
        c2 = pltpu.async_copy(t_hbm.at[pl.ds(base, RPW)], t_v, sem)
        c3 = pltpu.async_copy(g_hbm.at[pl.ds(base, RPW)], g_v, sem)
        c1.wait()
        c2.wait()
        c3.wait()

        J = range(CPI)

        def row_body(i, acc):
            ms = [s_v[i, pl.ds(j * L, L)] for j in J]
            mt = [t_v[i, pl.ds(j * L, L)] for j in J]
            for jj in range(CPI, NCHUNK, CPI):
                svs = [s_v[i, pl.ds((jj + j) * L, L)] for j in J]
                tvs = [t_v[i, pl.ds((jj + j) * L, L)] for j in J]
                ms = [jnp.maximum(a, b) for a, b in zip(ms, svs)]
                mt = [jnp.maximum(a, b) for a, b in zip(mt, tvs)]
            M_s, M_t = _bfly_multi(
                [functools.reduce(jnp.maximum, ms),
                 functools.reduce(jnp.maximum, mt)],
                [jnp.maximum, jnp.maximum],
            )

            zero = jnp.zeros((L,), jnp.float32)
            Es, Et, Av = zero, zero, zero
            for jj in range(0, NCHUNK, CPI):
                sls = [pl.ds((jj + j) * L, L) for j in J]
                svs = [s_v[i, sl] for sl in sls]
                tvs = [t_v[i, sl] for sl in sls]
                gvs = [g_v[i, sl] for sl in sls]
                xs = [sv - M_s for sv in svs]
                xt = [tv - M_t for tv in tvs]
                ess = [jnp.exp(x) for x in xs]
                ets = [jnp.exp(x) for x in xt]
                ds = [tv - sv for sv, tv in zip(svs, tvs)]
                gms = [1.0 - gv for gv in gvs]
                for j in J:
                    es_v[sls[j]] = ess[j]
                    et_v[sls[j]] = ets[j]
                    d_v[sls[j]] = ds[j]
                etgs = [et * gm for et, gm in zip(ets, gms)]
                for j in J:
                    Es = Es + ess[j] * gms[j]
                    Et = Et + etgs[j]
                    Av = Av + etgs[j] * ds[j]
            Es, Et, A = _bfly_multi([Es, Et, Av], [jnp.add] * 3)
            Kv = (M_s - M_t) - jnp.float32(127.0 * _LN2)

            kacc = zero
            for jj in range(0, NCHUNK, CPI):
                sls = [pl.ds((jj + j) * L, L) for j in J]
                ess = [es_v[sl] for sl in sls]
                ets = [et_v[sl] for sl in sls]
                ds = [d_v[sl] for sl in sls]
                zss = [Es + es for es in ess]
                zts = [Et + et for et in ets]
                nums = [A + et * d for et, d in zip(ets, ds)]
                rzts = [1.0 / zt for zt in zts]
                largs = [zs * rzt for zs, rzt in zip(zss, rzts)]
                logs = [_softlog_biased(la) for la in largs]
                for j in J:
                    kl = nums[j] * rzts[j] + logs[j] + Kv
                    kacc = kacc + kl * g_v[i, sls[j]]
            return acc + kacc

        acc = lax.fori_loop(0, RPW, row_body, jnp.zeros((L,), jnp.float32))
        acc_v[...] = acc * jnp.float32(1.0 / N)
        pltpu.sync_copy(acc_v, out_hbm.at[wid])

    return body(student, teacher, target)


def kernel(student, teacher, target):
    partials = _sc_partials(student, teacher, target)
    return jnp.sum(partials)


# speedup vs baseline: 1.1783x; 1.0016x over previous
import functools

import jax
import jax.numpy as jnp
from jax import lax
from jax.experimental import pallas as pl
from jax.experimental.pallas import tpu as pltpu
from jax.experimental.pallas import tpu_sc as plsc

N, C = 512, 256
NUM_CORES = 2
NUM_SUBCORES = 16
NW = NUM_CORES * NUM_SUBCORES
RPW = N // NW
L = 16
NCHUNK = C // L
CPI = 16

_LN2 = 0.6931471805599453
_LOG_COEF = (
    0.9999974966049194,
    -0.4999113082885742,
    0.3336314857006073,
    -0.2553149163722992,
    0.202413871884346,
    -0.10462728142738342,
)


def _softlog_biased(x):
    bits = plsc.bitcast(x, jnp.int32)
    eb = (bits + 0x00400000) >> 23
    scale = plsc.bitcast((254 - eb) << 23, jnp.float32)
    t = x * scale - 1.0
    p = jnp.full_like(t, _LOG_COEF[-1])
    for coef in _LOG_COEF[-2::-1]:
        p = p * t + coef
    return eb.astype(jnp.float32) * _LN2 + t * p


def _tree(vals, op):
    vals = list(vals)
    while len(vals) > 1:
        nxt = [op(vals[k], vals[k + 1]) for k in range(0, len(vals) - 1, 2)]
        if len(vals) % 2:
            nxt.append(vals[-1])
        vals = nxt
    return vals[0]


def _bfly_multi(vs, ops):
    for k in (8, 4, 2, 1):
        idx = lax.iota(jnp.int32, L) ^ k
        perms = [jnp.take_along_axis(v, idx, axis=0) for v in vs]
        vs = [op(v, pv) for v, pv, op in zip(vs, perms, ops)]
    return vs


def _sc_partials(student, teacher, target):
    mesh = plsc.VectorSubcoreMesh(core_axis_name="c", subcore_axis_name="s")

    @functools.partial(
        pl.kernel,
        out_type=jax.ShapeDtypeStruct((NW, L), jnp.float32),
        mesh=mesh,
        compiler_params=pltpu.CompilerParams(needs_layout_passes=False),
        scratch_types=[
            pltpu.VMEM((RPW, C), jnp.float32),
            pltpu.VMEM((RPW, C), jnp.float32),
            pltpu.VMEM((RPW, C), jnp.float32),
            pltpu.VMEM((C,), jnp.float32),
            pltpu.VMEM((C,), jnp.float32),
            pltpu.VMEM((C,), jnp.float32),
            pltpu.VMEM((L,), jnp.float32),
            pltpu.SemaphoreType.DMA,
        ],
    )
    def body(s_hbm, t_hbm, g_hbm, out_hbm, s_v, t_v, g_v, es_v, et_v, d_v,
             acc_v, sem):
        wid = lax.axis_index("s") * NUM_CORES + lax.axis_index("c")
        base = wid * RPW
        c1 = pltpu.async_copy(s_hbm.at[pl.ds(base, RPW)], s_v, sem)
        c2 = pltpu.async_copy(t_hbm.at[pl.ds(base, RPW)], t_v, sem)
        c3 = pltpu.async_copy(g_hbm.at[pl.ds(base, RPW)], g_v, sem)
        c1.wait()
        c2.wait()
        c3.wait()

        J = range(CPI)

        def row_body(i, acc):
            ms = [s_v[i, pl.ds(j * L, L)] for j in J]
            mt = [t_v[i, pl.ds(j * L, L)] for j in J]
            for jj in range(CPI, NCHUNK, CPI):
                svs = [s_v[i, pl.ds((jj + j) * L, L)] for j in J]
                tvs = [t_v[i, pl.ds((jj + j) * L, L)] for j in J]
                ms = [jnp.maximum(a, b) for a, b in zip(ms, svs)]
                mt = [jnp.maximum(a, b) for a, b in zip(mt, tvs)]
            M_s, M_t = _bfly_multi(
                [_tree(ms, jnp.maximum), _tree(mt, jnp.maximum)],
                [jnp.maximum, jnp.maximum],
            )

            zero = jnp.zeros((L,), jnp.float32)
            Es, Et, Av = zero, zero, zero
            for jj in range(0, NCHUNK, CPI):
                sls = [pl.ds((jj + j) * L, L) for j in J]
                svs = [s_v[i, sl] for sl in sls]
                tvs = [t_v[i, sl] for sl in sls]
                gvs = [g_v[i, sl] for sl in sls]
                xs = [sv - M_s for sv in svs]
                xt = [tv - M_t for tv in tvs]
                ess = [jnp.exp(x) for x in xs]
                ets = [jnp.exp(x) for x in xt]
                ds = [tv - sv for sv, tv in zip(svs, tvs)]
                gms = [1.0 - gv for gv in gvs]
                for j in J:
                    es_v[sls[j]] = ess[j]
                    et_v[sls[j]] = ets[j]
                    d_v[sls[j]] = ds[j]
                etgs = [et * gm for et, gm in zip(ets, gms)]
                for j in J:
                    Es = Es + ess[j] * gms[j]
                    Et = Et + etgs[j]
                    Av = Av + etgs[j] * ds[j]
            Es, Et, A = _bfly_multi([Es, Et, Av], [jnp.add] * 3)
            Kv = (M_s - M_t) - jnp.float32(127.0 * _LN2)

            kacc = zero
            for jj in range(0, NCHUNK, CPI):
                sls = [pl.ds((jj + j) * L, L) for j in J]
                ess = [es_v[sl] for sl in sls]
                ets = [et_v[sl] for sl in sls]
                ds = [d_v[sl] for sl in sls]
                zss = [Es + es for es in ess]
                zts = [Et + et for et in ets]
                nums = [A + et * d for et, d in zip(ets, ds)]
                rzts = [1.0 / zt for zt in zts]
                largs = [zs * rzt for zs, rzt in zip(zss, rzts)]
                logs = [_softlog_biased(la) for la in largs]
                for j in J:
                    kl = nums[j] * rzts[j] + logs[j] + Kv
                    kacc = kacc + kl * g_v[i, sls[j]]
            return acc + kacc

        acc = lax.fori_loop(0, RPW, row_body, jnp.zeros((L,), jnp.float32))
        acc_v[...] = acc * jnp.float32(1.0 / N)
        pltpu.sync_copy(acc_v, out_hbm.at[wid])

    return body(student, teacher, target)


def kernel(student, teacher, target):
    partials = _sc_partials(student, teacher, target)
    return jnp.sum(partials)


# speedup vs baseline: 1.1789x; 1.0005x over previous
import functools

import jax
import jax.numpy as jnp
from jax import lax
from jax.experimental import pallas as pl
from jax.experimental.pallas import tpu as pltpu
from jax.experimental.pallas import tpu_sc as plsc

N, C = 512, 256
N_SC = 256
N_TC = N - N_SC
NUM_CORES = 2
NUM_SUBCORES = 16
NW = NUM_CORES * NUM_SUBCORES
RPW = N_SC // NW
L = 16
NCHUNK = C // L
CPI = 16

_LN2 = 0.6931471805599453
_LOG_COEF = (
    0.9999974966049194,
    -0.4999113082885742,
    0.3336314857006073,
    -0.2553149163722992,
    0.202413871884346,
    -0.10462728142738342,
)


def _softlog_biased(x):
    bits = plsc.bitcast(x, jnp.int32)
    eb = (bits + 0x00400000) >> 23
    scale = plsc.bitcast((254 - eb) << 23, jnp.float32)
    t = x * scale - 1.0
    p = jnp.full_like(t, _LOG_COEF[-1])
    for coef in _LOG_COEF[-2::-1]:
        p = p * t + coef
    return eb.astype(jnp.float32) * _LN2 + t * p


def _tree(vals, op):
    vals = list(vals)
    while len(vals) > 1:
        nxt = [op(vals[k], vals[k + 1]) for k in range(0, len(vals) - 1, 2)]
        if len(vals) % 2:
            nxt.append(vals[-1])
        vals = nxt
    return vals[0]


def _bfly_multi(vs, ops):
    for k in (8, 4, 2, 1):
        idx = lax.iota(jnp.int32, L) ^ k
        perms = [jnp.take_along_axis(v, idx, axis=0) for v in vs]
        vs = [op(v, pv) for v, pv, op in zip(vs, perms, ops)]
    return vs


def _sc_partials(student, teacher, target):
    mesh = plsc.VectorSubcoreMesh(core_axis_name="c", subcore_axis_name="s")

    @functools.partial(
        pl.kernel,
        out_type=jax.ShapeDtypeStruct((NW, L), jnp.float32),
        mesh=mesh,
        compiler_params=pltpu.CompilerParams(needs_layout_passes=False),
        scratch_types=[
            pltpu.VMEM((RPW, C), jnp.float32),
            pltpu.VMEM((RPW, C), jnp.float32),
            pltpu.VMEM((RPW, C), jnp.float32),
            pltpu.VMEM((C,), jnp.float32),
            pltpu.VMEM((C,), jnp.float32),
            pltpu.VMEM((C,), jnp.float32),
            pltpu.VMEM((L,), jnp.float32),
            pltpu.SemaphoreType.DMA,
        ],
    )
    def body(s_hbm, t_hbm, g_hbm, out_hbm, s_v, t_v, g_v, es_v, et_v, d_v,
             acc_v, sem):
        wid = lax.axis_index("s") * NUM_CORES + lax.axis_index("c")
        base = wid * RPW
        c1 = pltpu.async_copy(s_hbm.at[pl.ds(base, RPW)], s_v, sem)
        c2 = pltpu.async_copy(t_hbm.at[pl.ds(base, RPW)], t_v, sem)
        c3 = pltpu.async_copy(g_hbm.at[pl.ds(base, RPW)], g_v, sem)
        c1.wait()
        c2.wait()
        c3.wait()

        J = range(CPI)

        def row_body(i, acc):
            ms = [s_v[i, pl.ds(j * L, L)] for j in J]
            mt = [t_v[i, pl.ds(j * L, L)] for j in J]
            for jj in range(CPI, NCHUNK, CPI):
                svs = [s_v[i, pl.ds((jj + j) * L, L)] for j in J]
                tvs = [t_v[i, pl.ds((jj + j) * L, L)] for j in J]
                ms = [jnp.maximum(a, b) for a, b in zip(ms, svs)]
                mt = [jnp.maximum(a, b) for a, b in zip(mt, tvs)]
            M_s, M_t = _bfly_multi(
                [_tree(ms, jnp.maximum), _tree(mt, jnp.maximum)],
                [jnp.maximum, jnp.maximum],
            )

            zero = jnp.zeros((L,), jnp.float32)
            Es, Et, Av = zero, zero, zero
            for jj in range(0, NCHUNK, CPI):
                sls = [pl.ds((jj + j) * L, L) for j in J]
                svs = [s_v[i, sl] for sl in sls]
                tvs = [t_v[i, sl] for sl in sls]
                gvs = [g_v[i, sl] for sl in sls]
                xs = [sv - M_s for sv in svs]
                xt = [tv - M_t for tv in tvs]
                ess = [jnp.exp(x) for x in xs]
                ets = [jnp.exp(x) for x in xt]
                ds = [tv - sv for sv, tv in zip(svs, tvs)]
                gms = [1.0 - gv for gv in gvs]
                for j in J:
                    es_v[sls[j]] = ess[j]
                    et_v[sls[j]] = ets[j]
                    d_v[sls[j]] = ds[j]
                etgs = [et * gm for et, gm in zip(ets, gms)]
                for j in J:
                    Es = Es + ess[j] * gms[j]
                    Et = Et + etgs[j]
                    Av = Av + etgs[j] * ds[j]
            Es, Et, A = _bfly_multi([Es, Et, Av], [jnp.add] * 3)
            Kv = (M_s - M_t) - jnp.float32(127.0 * _LN2)

            kacc = zero
            for jj in range(0, NCHUNK, CPI):
                sls = [pl.ds((jj + j) * L, L) for j in J]
                ess = [es_v[sl] for sl in sls]
                ets = [et_v[sl] for sl in sls]
                ds = [d_v[sl] for sl in sls]
                zss = [Es + es for es in ess]
                zts = [Et + et for et in ets]
                nums = [A + et * d for et, d in zip(ets, ds)]
                rzts = [1.0 / zt for zt in zts]
                largs = [zs * rzt for zs, rzt in zip(zss, rzts)]
                logs = [_softlog_biased(la) for la in largs]
                for j in J:
                    kl = nums[j] * rzts[j] + logs[j] + Kv
                    kacc = kacc + kl * g_v[i, sls[j]]
            return acc + kacc

        acc = lax.fori_loop(0, RPW, row_body, jnp.zeros((L,), jnp.float32))
        acc_v[...] = acc * jnp.float32(1.0 / N)
        pltpu.sync_copy(acc_v, out_hbm.at[wid])

    return body(student, teacher, target)


def _tc_body(s_ref, t_ref, g_ref, out_ref):
    s = s_ref[...]
    t = t_ref[...]
    g = g_ref[...]
    gm = 1.0 - g
    M_s = jnp.max(s, axis=1, keepdims=True)
    M_t = jnp.max(t, axis=1, keepdims=True)
    es = jnp.exp(s - M_s)
    et = jnp.exp(t - M_t)
    d = t - s
    Es = jnp.sum(es * gm, axis=1, keepdims=True)
    Et = jnp.sum(et * gm, axis=1, keepdims=True)
    A = jnp.sum(et * d * gm, axis=1, keepdims=True)
    rzt = 1.0 / (Et + et)
    kl = (A + et * d) * rzt + jnp.log((Es + es) * rzt) + (M_s - M_t)
    out_ref[0, 0] = jnp.sum(kl * g) * jnp.float32(1.0 / N)


def _tc_partial(student, teacher, target):
    spec = pl.BlockSpec((N_TC, C), lambda i: (1, 0))
    return pl.pallas_call(
        _tc_body,
        grid=(1,),
        in_specs=[spec, spec, spec],
        out_specs=pl.BlockSpec((1, 1), lambda i: (0, 0),
                               memory_space=pltpu.SMEM),
        out_shape=jax.ShapeDtypeStruct((1, 1), jnp.float32),
    )(student, teacher, target)


def kernel(student, teacher, target):
    sc = _sc_partials(student, teacher, target)
    tc = _tc_partial(student, teacher, target)
    return jnp.sum(sc) + tc[0, 0]


# speedup vs baseline: 1.1799x; 1.0008x over previous
import functools

import jax
import jax.numpy as jnp
from jax import lax
from jax.experimental import pallas as pl
from jax.experimental.pallas import tpu as pltpu
from jax.experimental.pallas import tpu_sc as plsc

N, C = 512, 256
NUM_CORES = 2
NUM_SUBCORES = 16
NW = NUM_CORES * NUM_SUBCORES
RPW = N // NW
L = 16
NCHUNK = C // L
CPI = 16

_LN2 = 0.6931471805599453
_LOG_COEF = (
    0.9999974966049194,
    -0.4999113082885742,
    0.3336314857006073,
    -0.2553149163722992,
    0.202413871884346,
    -0.10462728142738342,
)


def _softlog_biased(x):
    bits = plsc.bitcast(x, jnp.int32)
    eb = (bits + 0x00400000) >> 23
    scale = plsc.bitcast((254 - eb) << 23, jnp.float32)
    t = x * scale - 1.0
    p = jnp.full_like(t, _LOG_COEF[-1])
    for coef in _LOG_COEF[-2::-1]:
        p = p * t + coef
    return eb.astype(jnp.float32) * _LN2 + t * p


def _tree(vals, op):
    vals = list(vals)
    while len(vals) > 1:
        nxt = [op(vals[k], vals[k + 1]) for k in range(0, len(vals) - 1, 2)]
        if len(vals) % 2:
            nxt.append(vals[-1])
        vals = nxt
    return vals[0]


def _bfly_multi(vs, ops):
    for k in (8, 4, 2, 1):
        idx = lax.iota(jnp.int32, L) ^ k
        perms = [jnp.take_along_axis(v, idx, axis=0) for v in vs]
        vs = [op(v, pv) for v, pv, op in zip(vs, perms, ops)]
    return vs


def _sc_partials(student, teacher, target):
    mesh = plsc.VectorSubcoreMesh(core_axis_name="c", subcore_axis_name="s")

    @functools.partial(
        pl.kernel,
        out_type=jax.ShapeDtypeStruct((NW, L), jnp.float32),
        mesh=mesh,
        compiler_params=pltpu.CompilerParams(needs_layout_passes=False),
        scratch_types=[
            pltpu.VMEM((RPW, C), jnp.float32),
            pltpu.VMEM((RPW, C), jnp.float32),
            pltpu.VMEM((RPW, C), jnp.float32),
            pltpu.VMEM((C,), jnp.float32),
            pltpu.VMEM((C,), jnp.float32),
            pltpu.VMEM((C,), jnp.float32),
            pltpu.VMEM((L,), jnp.float32),
            pltpu.SemaphoreType.DMA,
        ],
    )
    def body(s_hbm, t_hbm, g_hbm, out_hbm, s_v, t_v, g_v, es_v, et_v, d_v,
             acc_v, sem):
        wid = lax.axis_index("s") * NUM_CORES + lax.axis_index("c")
        base = wid * RPW
        c1 = pltpu.async_copy(s_hbm.at[pl.ds(base, RPW)], s_v, sem)
        c2 = pltpu.async_copy(t_hbm.at[pl.ds(base, RPW)], t_v, sem)
        c3 = pltpu.async_copy(g_hbm.at[pl.ds(base, RPW)], g_v, sem)
        c1.wait()
        c2.wait()
        c3.wait()

        J = range(CPI)

        def row_body(i, acc):
            ms = [s_v[i, pl.ds(j * L, L)] for j in J]
            mt = [t_v[i, pl.ds(j * L, L)] for j in J]
            for jj in range(CPI, NCHUNK, CPI):
                svs = [s_v[i, pl.ds((jj + j) * L, L)] for j in J]
                tvs = [t_v[i, pl.ds((jj + j) * L, L)] for j in J]
                ms = [jnp.maximum(a, b) for a, b in zip(ms, svs)]
                mt = [jnp.maximum(a, b) for a, b in zip(mt, tvs)]
            M_s, M_t = _bfly_multi(
                [_tree(ms, jnp.maximum), _tree(mt, jnp.maximum)],
                [jnp.maximum, jnp.maximum],
            )

            zero = jnp.zeros((L,), jnp.float32)
            Es, Et, Av = zero, zero, zero
            for jj in range(0, NCHUNK, CPI):
                sls = [pl.ds((jj + j) * L, L) for j in J]
                svs = [s_v[i, sl] for sl in sls]
                tvs = [t_v[i, sl] for sl in sls]
                gvs = [g_v[i, sl] for sl in sls]
                xs = [sv - M_s for sv in svs]
                xt = [tv - M_t for tv in tvs]
                ess = [jnp.exp(x) for x in xs]
                ets = [jnp.exp(x) for x in xt]
                ds = [tv - sv for sv, tv in zip(svs, tvs)]
                gms = [1.0 - gv for gv in gvs]
                for j in J:
                    es_v[sls[j]] = ess[j]
                    et_v[sls[j]] = ets[j]
                    d_v[sls[j]] = ds[j]
                etgs = [et * gm for et, gm in zip(ets, gms)]
                for j in J:
                    Es = Es + ess[j] * gms[j]
                    Et = Et + etgs[j]
                    Av = Av + etgs[j] * ds[j]
            Es, Et, A = _bfly_multi([Es, Et, Av], [jnp.add] * 3)
            Kv = (M_s - M_t) - jnp.float32(127.0 * _LN2)

            kacc = zero
            for jj in range(0, NCHUNK, CPI):
                sls = [pl.ds((jj + j) * L, L) for j in J]
                ess = [es_v[sl] for sl in sls]
                ets = [et_v[sl] for sl in sls]
                ds = [d_v[sl] for sl in sls]
                zss = [Es + es for es in ess]
                zts = [Et + et for et in ets]
                nums = [A + et * d for et, d in zip(ets, ds)]
                rzts = [1.0 / zt for zt in zts]
                largs = [zs * rzt for zs, rzt in zip(zss, rzts)]
                logs = [_softlog_biased(la) for la in largs]
                for j in J:
                    kl = nums[j] * rzts[j] + logs[j] + Kv
                    kacc = kacc + kl * g_v[i, sls[j]]
            return acc + kacc

        acc = lax.fori_loop(0, RPW, row_body, jnp.zeros((L,), jnp.float32))
        acc_v[...] = acc * jnp.float32(1.0 / N)
        pltpu.sync_copy(acc_v, out_hbm.at[wid])

    return body(student, teacher, target)


def kernel(student, teacher, target):
    partials = _sc_partials(student, teacher, target)
    return jnp.sum(partials)
